# baseline (speedup 1.0000x reference)
import functools

import jax
import jax.numpy as jnp
from jax import lax
from jax.experimental import pallas as pl
from jax.experimental.pallas import tpu as pltpu
from jax.experimental.pallas import tpu_sc as plsc

EMBED = 128
CHUNK = 128
SEQ = 512
EPS = 1e-12
B_BLK = 32
NBUF = 4
NSLICE = 8


def _make_sc_gather(nrows, nworkers):
    rows_per_w = nrows // nworkers
    nchunks = rows_per_w // CHUNK
    mesh = plsc.VectorSubcoreMesh(core_axis_name="c", subcore_axis_name="s")

    @functools.partial(
        pl.kernel,
        mesh=mesh,
        out_type=jax.ShapeDtypeStruct((nrows, EMBED), jnp.float32),
        scratch_types=[
            pltpu.VMEM((rows_per_w,), jnp.int32)]
            + [pltpu.VMEM((CHUNK, EMBED), jnp.float32) for _ in range(NBUF)]
            + [pltpu.SemaphoreType.DMA for _ in range(NBUF)],
    )
    def k(word_hbm, ids_hbm, out_hbm, idx_v, *bufs_sems):
        rows = bufs_sems[:NBUF]
        sems = bufs_sems[NBUF:]
        wid = lax.axis_index("s") * 2 + lax.axis_index("c")
        wbase = wid * rows_per_w

        pltpu.sync_copy(ids_hbm.at[pl.ds(wbase, rows_per_w)], idx_v)

        def group_body(q, _):
            base = q * NBUF
            handles = []
            for i in range(NBUF):
                off = (base + i) * CHUNK
                handles.append(pltpu.async_copy(
                    word_hbm.at[idx_v.at[pl.ds(off, CHUNK)]], rows[i], sems[i]))
            for i in range(NBUF):
                off = (base + i) * CHUNK
                handles[i].wait()
                pltpu.sync_copy(rows[i], out_hbm.at[pl.ds(wbase + off, CHUNK)])
            return 0

        lax.fori_loop(0, nchunks // NBUF, group_body, 0)

    return k


def _ln_tc_kernel(g_ref, tt_ref, lo_ref, dt_ref, gam_ref, bet_ref, o_ref):
    x = g_ref[...]
    tt = tt_ref[...]
    lo = lo_ref[...]
    dt = dt_ref[...]
    x = x + lo[None, :, :] + tt[:, :, None] * dt[0][None, None, :]
    m = jnp.mean(x, axis=-1, keepdims=True)
    xc = x - m
    var = jnp.mean(xc * xc, axis=-1, keepdims=True)
    y = xc * lax.rsqrt(var + EPS) * gam_ref[0][None, None, :] + bet_ref[0][None, None, :]
    o_ref[...] = y


@jax.jit
def kernel(input_ids, token_type_ids, word_table, pos_table, type_table, gamma, beta):
    batch, seq = input_ids.shape
    nrows = batch * seq
    ids = input_ids.reshape(nrows).astype(jnp.int32)
    tt = token_type_ids.astype(jnp.float32)
    lo = pos_table + type_table[0]
    dt = (type_table[1] - type_table[0]).reshape(1, EMBED)
    gam = gamma.reshape(1, EMBED)
    bet = beta.reshape(1, EMBED)

    slice_batches = [batch // 8, 3 * batch // 8, 3 * batch // 8, batch // 8]
    sc_cache = {}

    data_specs = [
        pl.BlockSpec((B_BLK, seq, EMBED), lambda i: (i, 0, 0)),
        None,
        pl.BlockSpec((seq, EMBED), lambda i: (0, 0)),
        pl.BlockSpec((1, EMBED), lambda i: (0, 0)),
        pl.BlockSpec((1, EMBED), lambda i: (0, 0)),
        pl.BlockSpec((1, EMBED), lambda i: (0, 0)),
    ]
    out_shape = jax.ShapeDtypeStruct((batch, seq, EMBED), jnp.float32)

    acc = None
    ob = 0
    for b_sl in slice_batches:
        rows_sl = b_sl * seq
        blocks_sl = b_sl // B_BLK
        blk0 = ob // B_BLK
        if rows_sl not in sc_cache:
            sc_cache[rows_sl] = _make_sc_gather(rows_sl, 32)
        g_s = sc_cache[rows_sl](word_table,
                                lax.dynamic_slice_in_dim(ids, ob * seq, rows_sl))
        g3 = g_s.reshape(b_sl, seq, EMBED)
        specs = list(data_specs)
        specs[1] = pl.BlockSpec((B_BLK, seq), lambda i, b=blk0: (b + i, 0))
        out_spec = pl.BlockSpec((B_BLK, seq, EMBED),
                                lambda i, b=blk0: (b + i, 0, 0))
        ob += b_sl
        operands = (g3, tt, lo, dt, gam, bet)
        if acc is None:
            acc = pl.pallas_call(
                _ln_tc_kernel, grid=(blocks_sl,), in_specs=specs,
                out_specs=out_spec, out_shape=out_shape,
            )(*operands)
        else:
            acc = pl.pallas_call(
                lambda a, *args: _ln_tc_kernel(*args),
                grid=(blocks_sl,),
                in_specs=[pl.BlockSpec(memory_space=pl.ANY)] + specs,
                out_specs=out_spec, out_shape=out_shape,
                input_output_aliases={0: 0},
            )(acc, *operands)
    return acc


# speedup vs baseline: 1.0522x; 1.0522x over previous
import functools

import jax
import jax.numpy as jnp
from jax import lax
from jax.experimental import pallas as pl
from jax.experimental.pallas import tpu as pltpu
from jax.experimental.pallas import tpu_sc as plsc

EMBED = 128
CHUNK = 128
SEQ = 512
EPS = 1e-12
B_BLK = 32
NBUF = 6
NSLICE = 8


def _make_sc_gather(nrows, nworkers):
    rows_per_w = nrows // nworkers
    nchunks = rows_per_w // CHUNK
    mesh = plsc.VectorSubcoreMesh(core_axis_name="c", subcore_axis_name="s")

    @functools.partial(
        pl.kernel,
        mesh=mesh,
        out_type=jax.ShapeDtypeStruct((nrows, EMBED), jnp.float32),
        scratch_types=[
            pltpu.VMEM((rows_per_w,), jnp.int32)]
            + [pltpu.VMEM((CHUNK, EMBED), jnp.float32) for _ in range(NBUF)]
            + [pltpu.SemaphoreType.DMA for _ in range(NBUF)],
    )
    def k(word_hbm, ids_hbm, out_hbm, idx_v, *bufs_sems):
        rows = bufs_sems[:NBUF]
        sems = bufs_sems[NBUF:]
        wid = lax.axis_index("s") * 2 + lax.axis_index("c")
        wbase = wid * rows_per_w

        pltpu.sync_copy(ids_hbm.at[pl.ds(wbase, rows_per_w)], idx_v)

        def group_body(q, _):
            base = q * NBUF
            handles = []
            for i in range(NBUF):
                off = (base + i) * CHUNK
                handles.append(pltpu.async_copy(
                    word_hbm.at[idx_v.at[pl.ds(off, CHUNK)]], rows[i], sems[i]))
            for i in range(NBUF):
                off = (base + i) * CHUNK
                handles[i].wait()
                pltpu.sync_copy(rows[i], out_hbm.at[pl.ds(wbase + off, CHUNK)])
            return 0

        lax.fori_loop(0, nchunks // NBUF, group_body, 0)

    return k


def _ln_tc_kernel(g_ref, tt_ref, lo_ref, dt_ref, gam_ref, bet_ref, o_ref):
    x = g_ref[...]
    tt = tt_ref[...]
    lo = lo_ref[...]
    dt = dt_ref[...]
    x = x + lo[None, :, :] + tt[:, :, None] * dt[0][None, None, :]
    m = jnp.mean(x, axis=-1, keepdims=True)
    xc = x - m
    var = jnp.mean(xc * xc, axis=-1, keepdims=True)
    y = xc * lax.rsqrt(var + EPS) * gam_ref[0][None, None, :] + bet_ref[0][None, None, :]
    o_ref[...] = y


@jax.jit
def kernel(input_ids, token_type_ids, word_table, pos_table, type_table, gamma, beta):
    batch, seq = input_ids.shape
    nrows = batch * seq
    ids = input_ids.reshape(nrows).astype(jnp.int32)
    tt = token_type_ids.astype(jnp.float32)
    lo = pos_table + type_table[0]
    dt = (type_table[1] - type_table[0]).reshape(1, EMBED)
    gam = gamma.reshape(1, EMBED)
    bet = beta.reshape(1, EMBED)

    slice_batches = [batch // 4] * 4
    sc_cache = {}

    data_specs = [
        pl.BlockSpec((B_BLK, seq, EMBED), lambda i: (i, 0, 0)),
        None,
        pl.BlockSpec((seq, EMBED), lambda i: (0, 0)),
        pl.BlockSpec((1, EMBED), lambda i: (0, 0)),
        pl.BlockSpec((1, EMBED), lambda i: (0, 0)),
        pl.BlockSpec((1, EMBED), lambda i: (0, 0)),
    ]
    out_shape = jax.ShapeDtypeStruct((batch, seq, EMBED), jnp.float32)

    acc = None
    ob = 0
    for b_sl in slice_batches:
        rows_sl = b_sl * seq
        blocks_sl = b_sl // B_BLK
        blk0 = ob // B_BLK
        if rows_sl not in sc_cache:
            sc_cache[rows_sl] = _make_sc_gather(rows_sl, 32)
        g_s = sc_cache[rows_sl](word_table,
                                lax.dynamic_slice_in_dim(ids, ob * seq, rows_sl))
        g3 = g_s.reshape(b_sl, seq, EMBED)
        specs = list(data_specs)
        specs[1] = pl.BlockSpec((B_BLK, seq), lambda i, b=blk0: (b + i, 0))
        out_spec = pl.BlockSpec((B_BLK, seq, EMBED),
                                lambda i, b=blk0: (b + i, 0, 0))
        ob += b_sl
        operands = (g3, tt, lo, dt, gam, bet)
        if acc is None:
            acc = pl.pallas_call(
                _ln_tc_kernel, grid=(blocks_sl,), in_specs=specs,
                out_specs=out_spec, out_shape=out_shape,
            )(*operands)
        else:
            acc = pl.pallas_call(
                lambda a, *args: _ln_tc_kernel(*args),
                grid=(blocks_sl,),
                in_specs=[pl.BlockSpec(memory_space=pl.ANY)] + specs,
                out_specs=out_spec, out_shape=out_shape,
                input_output_aliases={0: 0},
            )(acc, *operands)
    return acc


# speedup vs baseline: 1.0752x; 1.0219x over previous
import functools

import jax
import jax.numpy as jnp
from jax import lax
from jax.experimental import pallas as pl
from jax.experimental.pallas import tpu as pltpu
from jax.experimental.pallas import tpu_sc as plsc

EMBED = 128
CHUNK = 128
SEQ = 512
EPS = 1e-12
B_BLK = 32
NBUF = 7
NSLICE = 8


def _make_sc_gather(nrows, nworkers):
    rows_per_w = nrows // nworkers
    nchunks = rows_per_w // CHUNK
    mesh = plsc.VectorSubcoreMesh(core_axis_name="c", subcore_axis_name="s")

    @functools.partial(
        pl.kernel,
        mesh=mesh,
        out_type=jax.ShapeDtypeStruct((nrows, EMBED), jnp.float32),
        scratch_types=[
            pltpu.VMEM((rows_per_w,), jnp.int32)]
            + [pltpu.VMEM((CHUNK, EMBED), jnp.float32) for _ in range(NBUF)]
            + [pltpu.SemaphoreType.DMA for _ in range(NBUF)],
    )
    def k(word_hbm, ids_hbm, out_hbm, idx_v, *bufs_sems):
        rows = bufs_sems[:NBUF]
        sems = bufs_sems[NBUF:]
        wid = lax.axis_index("s") * 2 + lax.axis_index("c")
        wbase = wid * rows_per_w

        pltpu.sync_copy(ids_hbm.at[pl.ds(wbase, rows_per_w)], idx_v)

        def group_body(q, _):
            base = q * NBUF
            handles = []
            for i in range(NBUF):
                off = (base + i) * CHUNK
                handles.append(pltpu.async_copy(
                    word_hbm.at[idx_v.at[pl.ds(off, CHUNK)]], rows[i], sems[i]))
            for i in range(NBUF):
                off = (base + i) * CHUNK
                handles[i].wait()
                pltpu.sync_copy(rows[i], out_hbm.at[pl.ds(wbase + off, CHUNK)])
            return 0

        lax.fori_loop(0, nchunks // NBUF, group_body, 0)

    return k


def _ln_tc_kernel(g_ref, tt_ref, lo_ref, dt_ref, gam_ref, bet_ref, o_ref):
    x = g_ref[...]
    tt = tt_ref[...]
    lo = lo_ref[...]
    dt = dt_ref[...]
    x = x + lo[None, :, :] + tt[:, :, None] * dt[0][None, None, :]
    m = jnp.mean(x, axis=-1, keepdims=True)
    xc = x - m
    var = jnp.mean(xc * xc, axis=-1, keepdims=True)
    y = xc * lax.rsqrt(var + EPS) * gam_ref[0][None, None, :] + bet_ref[0][None, None, :]
    o_ref[...] = y


@jax.jit
def kernel(input_ids, token_type_ids, word_table, pos_table, type_table, gamma, beta):
    batch, seq = input_ids.shape
    nrows = batch * seq
    ids = input_ids.reshape(nrows).astype(jnp.int32)
    tt = token_type_ids.astype(jnp.float32)
    lo = pos_table + type_table[0]
    dt = (type_table[1] - type_table[0]).reshape(1, EMBED)
    gam = gamma.reshape(1, EMBED)
    bet = beta.reshape(1, EMBED)

    slice_batches = [batch // 4] * 4
    sc_cache = {}

    data_specs = [
        pl.BlockSpec((B_BLK, seq, EMBED), lambda i: (i, 0, 0)),
        None,
        pl.BlockSpec((seq, EMBED), lambda i: (0, 0)),
        pl.BlockSpec((1, EMBED), lambda i: (0, 0)),
        pl.BlockSpec((1, EMBED), lambda i: (0, 0)),
        pl.BlockSpec((1, EMBED), lambda i: (0, 0)),
    ]
    out_shape = jax.ShapeDtypeStruct((batch, seq, EMBED), jnp.float32)

    acc = None
    ob = 0
    for b_sl in slice_batches:
        rows_sl = b_sl * seq
        blocks_sl = b_sl // B_BLK
        blk0 = ob // B_BLK
        if rows_sl not in sc_cache:
            sc_cache[rows_sl] = _make_sc_gather(rows_sl, 32)
        g_s = sc_cache[rows_sl](word_table,
                                lax.dynamic_slice_in_dim(ids, ob * seq, rows_sl))
        g3 = g_s.reshape(b_sl, seq, EMBED)
        specs = list(data_specs)
        specs[1] = pl.BlockSpec((B_BLK, seq), lambda i, b=blk0: (b + i, 0))
        out_spec = pl.BlockSpec((B_BLK, seq, EMBED),
                                lambda i, b=blk0: (b + i, 0, 0))
        ob += b_sl
        operands = (g3, tt, lo, dt, gam, bet)
        if acc is None:
            acc = pl.pallas_call(
                _ln_tc_kernel, grid=(blocks_sl,), in_specs=specs,
                out_specs=out_spec, out_shape=out_shape,
            )(*operands)
        else:
            acc = pl.pallas_call(
                lambda a, *args: _ln_tc_kernel(*args),
                grid=(blocks_sl,),
                in_specs=[pl.BlockSpec(memory_space=pl.ANY)] + specs,
                out_specs=out_spec, out_shape=out_shape,
                input_output_aliases={0: 0},
            )(acc, *operands)
    return acc
